# Initial kernel scaffold; baseline (speedup 1.0000x reference)
import jax
import jax.numpy as jnp
from jax.experimental import pallas as pl


def kernel(node_feat, edge_feat, edge_index, w_n2l_W, w_n2l_b, w_e2l_W, w_e2l_b, conv_W, conv_b, out_W, out_b):
    raise NotImplementedError("write your pallas kernel here")


# speedup vs baseline: 5.4152x; 5.4152x over previous
import functools

import jax
import jax.numpy as jnp
from jax import lax
from jax.experimental import pallas as pl
from jax.experimental.pallas import tpu as pltpu
from jax.experimental.pallas import tpu_sc as plsc

N = 10000
E = 320000
DN = 128
DE = 16
MAX_LV = 3

NC = 2
NS = 16
NW = NC * NS
CH = 80
NCHT = E // (NW * CH)
EPT = E // NW
RPT = 624
TAIL = N - NS * RPT

_mesh = lambda: plsc.VectorSubcoreMesh(core_axis_name="c", subcore_axis_name="s")


@functools.partial(
    pl.kernel, mesh=_mesh(),
    out_type=jax.ShapeDtypeStruct((NC * N, DN), jnp.float32),
    scratch_types=[
        pltpu.VMEM((NCHT, CH), jnp.int32),
        pltpu.VMEM((CH, DE), jnp.float32),
        pltpu.VMEM((CH, DN), jnp.float32),
        pltpu.VMEM_SHARED((N, DN), jnp.float32),
        pltpu.SemaphoreType.DMA,
    ])
def _sc_seg16(ef_hbm, dst_hbm, zeros_hbm, out_hbm, dst_v, rows16_v, rows_v,
              accum, sem):
    c = lax.axis_index("c")
    s = lax.axis_index("s")
    w = c * NS + s
    pltpu.sync_copy(dst_hbm.at[w], dst_v)
    pltpu.sync_copy(zeros_hbm.at[pl.ds(0, CH)], rows_v)
    pltpu.sync_copy(zeros_hbm.at[pl.ds(s * RPT, RPT)],
                    accum.at[pl.ds(s * RPT, RPT)])

    @pl.when(s == NS - 1)
    def _():
        pltpu.sync_copy(zeros_hbm.at[pl.ds(NS * RPT, TAIL)],
                        accum.at[pl.ds(NS * RPT, TAIL)])

    plsc.subcore_barrier()

    def body(j, carry):
        pltpu.async_copy(ef_hbm.at[pl.ds(w * EPT + j * CH, CH)], rows16_v,
                         sem).wait()

        def pad(j2, carry2):
            rows_v[j2, pl.ds(0, DE)] = rows16_v[j2, :]
            return carry2

        lax.fori_loop(0, CH, pad, 0)
        pltpu.sync_copy(rows_v, accum.at[dst_v.at[j]], add=True)
        return carry

    lax.fori_loop(0, NCHT, body, 0)
    plsc.subcore_barrier()
    pltpu.sync_copy(accum.at[pl.ds(s * RPT, RPT)],
                    out_hbm.at[pl.ds(c * N + s * RPT, RPT)])

    @pl.when(s == NS - 1)
    def _():
        pltpu.sync_copy(accum.at[pl.ds(NS * RPT, TAIL)],
                        out_hbm.at[pl.ds(c * N + NS * RPT, TAIL)])


@functools.partial(
    pl.kernel, mesh=_mesh(),
    out_type=jax.ShapeDtypeStruct((NC * N, DN), jnp.float32),
    scratch_types=[
        pltpu.VMEM((NCHT, CH), jnp.int32),
        pltpu.VMEM((NCHT, CH), jnp.int32),
        pltpu.VMEM((CH, DN), jnp.float32),
        pltpu.VMEM_SHARED((N, DN), jnp.float32),
        pltpu.SemaphoreType.DMA,
    ])
def _sc_gather_segsum(cur_hbm, src_hbm, dst_hbm, zeros_hbm, out_hbm,
                      src_v, dst_v, rows_v, accum, sem):
    c = lax.axis_index("c")
    s = lax.axis_index("s")
    w = c * NS + s
    pltpu.sync_copy(src_hbm.at[w], src_v)
    pltpu.sync_copy(dst_hbm.at[w], dst_v)
    pltpu.sync_copy(zeros_hbm.at[pl.ds(s * RPT, RPT)],
                    accum.at[pl.ds(s * RPT, RPT)])

    @pl.when(s == NS - 1)
    def _():
        pltpu.sync_copy(zeros_hbm.at[pl.ds(NS * RPT, TAIL)],
                        accum.at[pl.ds(NS * RPT, TAIL)])

    plsc.subcore_barrier()

    def body(j, carry):
        pltpu.async_copy(cur_hbm.at[src_v.at[j]], rows_v, sem).wait()
        pltpu.sync_copy(rows_v, accum.at[dst_v.at[j]], add=True)
        return carry

    lax.fori_loop(0, NCHT, body, 0)
    plsc.subcore_barrier()
    pltpu.sync_copy(accum.at[pl.ds(s * RPT, RPT)],
                    out_hbm.at[pl.ds(c * N + s * RPT, RPT)])

    @pl.when(s == NS - 1)
    def _():
        pltpu.sync_copy(accum.at[pl.ds(NS * RPT, TAIL)],
                        out_hbm.at[pl.ds(c * N + NS * RPT, TAIL)])


_RB = 1000


def _tc_prelude_body(nf, s0, s1, w1, w2, b1, b2, im_ref, cur_ref):
    acc = jnp.dot(nf[...], w1[...], preferred_element_type=jnp.float32)
    seg = s0[...] + s1[...]
    acc = acc + jnp.dot(seg[:, :DE], w2[...],
                        preferred_element_type=jnp.float32)
    acc = acc + b1[...] + b2[...]
    im_ref[...] = acc
    cur_ref[...] = jnp.maximum(acc, 0.0)


_tc_prelude = pl.pallas_call(
    _tc_prelude_body,
    grid=(N // _RB,),
    in_specs=[
        pl.BlockSpec((_RB, DN), lambda i: (i, 0)),
        pl.BlockSpec((_RB, DN), lambda i: (i, 0)),
        pl.BlockSpec((_RB, DN), lambda i: (i, 0)),
        pl.BlockSpec((DN, DN), lambda i: (0, 0)),
        pl.BlockSpec((DE, DN), lambda i: (0, 0)),
        pl.BlockSpec((1, DN), lambda i: (0, 0)),
        pl.BlockSpec((1, DN), lambda i: (0, 0)),
    ],
    out_specs=[
        pl.BlockSpec((_RB, DN), lambda i: (i, 0)),
        pl.BlockSpec((_RB, DN), lambda i: (i, 0)),
    ],
    out_shape=[
        jax.ShapeDtypeStruct((N, DN), jnp.float32),
        jax.ShapeDtypeStruct((N, DN), jnp.float32),
    ],
)


def _tc_update_body(p0, p1, im, w, b, cur_ref):
    pool = p0[...] + p1[...]
    acc = jnp.dot(pool, w[...], preferred_element_type=jnp.float32)
    cur_ref[...] = jnp.maximum(acc + b[...] + im[...], 0.0)


_tc_update = pl.pallas_call(
    _tc_update_body,
    grid=(N // _RB,),
    in_specs=[
        pl.BlockSpec((_RB, DN), lambda i: (i, 0)),
        pl.BlockSpec((_RB, DN), lambda i: (i, 0)),
        pl.BlockSpec((_RB, DN), lambda i: (i, 0)),
        pl.BlockSpec((DN, DN), lambda i: (0, 0)),
        pl.BlockSpec((1, DN), lambda i: (0, 0)),
    ],
    out_specs=pl.BlockSpec((_RB, DN), lambda i: (i, 0)),
    out_shape=jax.ShapeDtypeStruct((N, DN), jnp.float32),
)


def _tc_out_body(cur, w, b, y_ref):
    i = pl.program_id(0)
    t = jnp.dot(cur[...], w[...], preferred_element_type=jnp.float32)
    t = jnp.maximum(t + b[...], 0.0)
    part = jnp.sum(t, axis=0, keepdims=True)

    @pl.when(i == 0)
    def _():
        y_ref[...] = part

    @pl.when(i > 0)
    def _():
        y_ref[...] = y_ref[...] + part

    @pl.when(i == pl.num_programs(0) - 1)
    def _():
        y_ref[...] = jnp.maximum(y_ref[...], 0.0)


_tc_out = pl.pallas_call(
    _tc_out_body,
    grid=(N // _RB,),
    in_specs=[
        pl.BlockSpec((_RB, DN), lambda i: (i, 0)),
        pl.BlockSpec((DN, DN), lambda i: (0, 0)),
        pl.BlockSpec((1, DN), lambda i: (0, 0)),
    ],
    out_specs=pl.BlockSpec((1, DN), lambda i: (0, 0)),
    out_shape=jax.ShapeDtypeStruct((1, DN), jnp.float32),
)


def kernel(node_feat, edge_feat, edge_index,
           w_n2l_W, w_n2l_b, w_e2l_W, w_e2l_b,
           conv_W, conv_b, out_W, out_b):
    edge_index = edge_index.astype(jnp.int32)
    src3d = edge_index[0].reshape(NW, NCHT, CH)
    dst3d = edge_index[1].reshape(NW, NCHT, CH)
    zeros128 = jnp.zeros((N, DN), jnp.float32)
    b_n2l = w_n2l_b.reshape(1, DN)
    b_e2l = w_e2l_b.reshape(1, DN)
    b_conv = conv_b.reshape(1, DN)
    b_out = out_b.reshape(1, DN)

    seg = _sc_seg16(edge_feat, dst3d, zeros128)
    im, cur = _tc_prelude(node_feat, seg[:N], seg[N:],
                          w_n2l_W, w_e2l_W, b_n2l, b_e2l)
    for _ in range(MAX_LV):
        pools = _sc_gather_segsum(cur, src3d, dst3d, zeros128)
        cur = _tc_update(pools[:N], pools[N:], im, conv_W, b_conv)
    return _tc_out(cur, out_W, b_out)


# speedup vs baseline: 7.1317x; 1.3170x over previous
import functools

import jax
import jax.numpy as jnp
from jax import lax
from jax.experimental import pallas as pl
from jax.experimental.pallas import tpu as pltpu
from jax.experimental.pallas import tpu_sc as plsc

N = 10000
E = 320000
DN = 128
DE = 16
MAX_LV = 3

NC = 2
NS = 16
NW = NC * NS
CH = 80
NCHT = E // (NW * CH)
EPT = E // NW
CH2 = 100
NBLK = 5
BCH = 20
NCHT2 = NBLK * BCH
RPT = 624
TAIL = N - NS * RPT

_mesh = lambda: plsc.VectorSubcoreMesh(core_axis_name="c", subcore_axis_name="s")


@functools.partial(
    pl.kernel, mesh=_mesh(),
    out_type=jax.ShapeDtypeStruct((NC * N, DN), jnp.float32),
    scratch_types=[
        pltpu.VMEM((NCHT, CH), jnp.int32),
        pltpu.VMEM((CH, DE), jnp.float32),
        pltpu.VMEM((CH, DN), jnp.float32),
        pltpu.VMEM_SHARED((N, DN), jnp.float32),
        pltpu.SemaphoreType.DMA,
    ])
def _sc_seg16(ef_hbm, dst_hbm, zeros_hbm, out_hbm, dst_v, rows16_v, rows_v,
              accum, sem):
    c = lax.axis_index("c")
    s = lax.axis_index("s")
    w = c * NS + s
    pltpu.sync_copy(dst_hbm.at[w], dst_v)
    pltpu.sync_copy(zeros_hbm.at[pl.ds(0, CH)], rows_v)
    pltpu.sync_copy(zeros_hbm.at[pl.ds(s * RPT, RPT)],
                    accum.at[pl.ds(s * RPT, RPT)])

    @pl.when(s == NS - 1)
    def _():
        pltpu.sync_copy(zeros_hbm.at[pl.ds(NS * RPT, TAIL)],
                        accum.at[pl.ds(NS * RPT, TAIL)])

    plsc.subcore_barrier()

    def body(j, carry):
        pltpu.async_copy(ef_hbm.at[pl.ds(w * EPT + j * CH, CH)], rows16_v,
                         sem).wait()

        def pad(j2, carry2):
            rows_v[j2, pl.ds(0, DE)] = rows16_v[j2, :]
            return carry2

        lax.fori_loop(0, CH, pad, 0)
        pltpu.sync_copy(rows_v, accum.at[dst_v.at[j]], add=True)
        return carry

    lax.fori_loop(0, NCHT, body, 0)
    plsc.subcore_barrier()
    pltpu.sync_copy(accum.at[pl.ds(s * RPT, RPT)],
                    out_hbm.at[pl.ds(c * N + s * RPT, RPT)])

    @pl.when(s == NS - 1)
    def _():
        pltpu.sync_copy(accum.at[pl.ds(NS * RPT, TAIL)],
                        out_hbm.at[pl.ds(c * N + NS * RPT, TAIL)])


@functools.partial(
    pl.kernel, mesh=_mesh(),
    out_type=jax.ShapeDtypeStruct((NC * N, DN), jnp.float32),
    scratch_types=[
        pltpu.VMEM((BCH, CH2), jnp.int32),
        pltpu.VMEM((BCH, CH2), jnp.int32),
        pltpu.VMEM((CH2, DN), jnp.float32),
        pltpu.VMEM((CH2, DN), jnp.float32),
        pltpu.VMEM_SHARED((N, DN), jnp.float32),
        pltpu.SemaphoreType.DMA,
        pltpu.SemaphoreType.DMA,
    ])
def _sc_gather_segsum(cur_hbm, src_hbm, dst_hbm, zeros_hbm, out_hbm,
                      src_v, dst_v, rows0, rows1, accum, sem0, sem1):
    c = lax.axis_index("c")
    s = lax.axis_index("s")
    w = c * NS + s
    pltpu.sync_copy(zeros_hbm.at[pl.ds(s * RPT, RPT)],
                    accum.at[pl.ds(s * RPT, RPT)])

    @pl.when(s == NS - 1)
    def _():
        pltpu.sync_copy(zeros_hbm.at[pl.ds(NS * RPT, TAIL)],
                        accum.at[pl.ds(NS * RPT, TAIL)])

    plsc.subcore_barrier()

    def blk_body(b, carry):
        pltpu.sync_copy(src_hbm.at[w * NBLK + b], src_v)
        pltpu.sync_copy(dst_hbm.at[w * NBLK + b], dst_v)
        pltpu.async_copy(cur_hbm.at[src_v.at[0]], rows0, sem0)

        def body(t, carry2):
            j0 = 2 * t
            j1 = j0 + 1
            j2 = j0 + 2
            pltpu.async_copy(cur_hbm.at[src_v.at[j1]], rows1, sem1)
            pltpu.make_async_copy(cur_hbm.at[src_v.at[j0]], rows0,
                                  sem0).wait()
            pltpu.sync_copy(rows0, accum.at[dst_v.at[j0]], add=True)

            @pl.when(j2 < BCH)
            def _():
                pltpu.async_copy(cur_hbm.at[src_v.at[j2]], rows0, sem0)

            pltpu.make_async_copy(cur_hbm.at[src_v.at[j1]], rows1,
                                  sem1).wait()
            pltpu.sync_copy(rows1, accum.at[dst_v.at[j1]], add=True)
            return carry2

        lax.fori_loop(0, BCH // 2, body, 0)
        return carry

    lax.fori_loop(0, NBLK, blk_body, 0)
    plsc.subcore_barrier()
    pltpu.sync_copy(accum.at[pl.ds(s * RPT, RPT)],
                    out_hbm.at[pl.ds(c * N + s * RPT, RPT)])

    @pl.when(s == NS - 1)
    def _():
        pltpu.sync_copy(accum.at[pl.ds(NS * RPT, TAIL)],
                        out_hbm.at[pl.ds(c * N + NS * RPT, TAIL)])


_RB = 1000


def _tc_prelude_body(nf, s0, s1, w1, w2, b1, b2, im_ref, cur_ref):
    acc = jnp.dot(nf[...], w1[...], preferred_element_type=jnp.float32)
    seg = s0[...] + s1[...]
    acc = acc + jnp.dot(seg[:, :DE], w2[...],
                        preferred_element_type=jnp.float32)
    acc = acc + b1[...] + b2[...]
    im_ref[...] = acc
    cur_ref[...] = jnp.maximum(acc, 0.0)


_tc_prelude = pl.pallas_call(
    _tc_prelude_body,
    grid=(N // _RB,),
    in_specs=[
        pl.BlockSpec((_RB, DN), lambda i: (i, 0)),
        pl.BlockSpec((_RB, DN), lambda i: (i, 0)),
        pl.BlockSpec((_RB, DN), lambda i: (i, 0)),
        pl.BlockSpec((DN, DN), lambda i: (0, 0)),
        pl.BlockSpec((DE, DN), lambda i: (0, 0)),
        pl.BlockSpec((1, DN), lambda i: (0, 0)),
        pl.BlockSpec((1, DN), lambda i: (0, 0)),
    ],
    out_specs=[
        pl.BlockSpec((_RB, DN), lambda i: (i, 0)),
        pl.BlockSpec((_RB, DN), lambda i: (i, 0)),
    ],
    out_shape=[
        jax.ShapeDtypeStruct((N, DN), jnp.float32),
        jax.ShapeDtypeStruct((N, DN), jnp.float32),
    ],
)


def _tc_update_body(p0, p1, im, w, b, cur_ref):
    pool = p0[...] + p1[...]
    acc = jnp.dot(pool, w[...], preferred_element_type=jnp.float32)
    cur_ref[...] = jnp.maximum(acc + b[...] + im[...], 0.0)


_tc_update = pl.pallas_call(
    _tc_update_body,
    grid=(N // _RB,),
    in_specs=[
        pl.BlockSpec((_RB, DN), lambda i: (i, 0)),
        pl.BlockSpec((_RB, DN), lambda i: (i, 0)),
        pl.BlockSpec((_RB, DN), lambda i: (i, 0)),
        pl.BlockSpec((DN, DN), lambda i: (0, 0)),
        pl.BlockSpec((1, DN), lambda i: (0, 0)),
    ],
    out_specs=pl.BlockSpec((_RB, DN), lambda i: (i, 0)),
    out_shape=jax.ShapeDtypeStruct((N, DN), jnp.float32),
)


def _tc_out_body(cur, w, b, y_ref):
    i = pl.program_id(0)
    t = jnp.dot(cur[...], w[...], preferred_element_type=jnp.float32)
    t = jnp.maximum(t + b[...], 0.0)
    part = jnp.sum(t, axis=0, keepdims=True)

    @pl.when(i == 0)
    def _():
        y_ref[...] = part

    @pl.when(i > 0)
    def _():
        y_ref[...] = y_ref[...] + part

    @pl.when(i == pl.num_programs(0) - 1)
    def _():
        y_ref[...] = jnp.maximum(y_ref[...], 0.0)


_tc_out = pl.pallas_call(
    _tc_out_body,
    grid=(N // _RB,),
    in_specs=[
        pl.BlockSpec((_RB, DN), lambda i: (i, 0)),
        pl.BlockSpec((DN, DN), lambda i: (0, 0)),
        pl.BlockSpec((1, DN), lambda i: (0, 0)),
    ],
    out_specs=pl.BlockSpec((1, DN), lambda i: (0, 0)),
    out_shape=jax.ShapeDtypeStruct((1, DN), jnp.float32),
)


def kernel(node_feat, edge_feat, edge_index,
           w_n2l_W, w_n2l_b, w_e2l_W, w_e2l_b,
           conv_W, conv_b, out_W, out_b):
    edge_index = edge_index.astype(jnp.int32)
    src3d = edge_index[0].reshape(NW * NBLK, BCH, CH2)
    dst3d = edge_index[1].reshape(NW * NBLK, BCH, CH2)
    dst3d80 = edge_index[1].reshape(NW, NCHT, CH)
    zeros128 = jnp.zeros((N, DN), jnp.float32)
    b_n2l = w_n2l_b.reshape(1, DN)
    b_e2l = w_e2l_b.reshape(1, DN)
    b_conv = conv_b.reshape(1, DN)
    b_out = out_b.reshape(1, DN)

    seg = _sc_seg16(edge_feat, dst3d80, zeros128)
    im, cur = _tc_prelude(node_feat, seg[:N], seg[N:],
                          w_n2l_W, w_e2l_W, b_n2l, b_e2l)
    for _ in range(MAX_LV):
        pools = _sc_gather_segsum(cur, src3d, dst3d, zeros128)
        cur = _tc_update(pools[:N], pools[N:], im, conv_W, b_conv)
    return _tc_out(cur, out_W, b_out)
